# baseline (speedup 1.0000x reference)
import functools

import jax
import jax.numpy as jnp
from jax import lax
from jax.experimental import pallas as pl
from jax.experimental.pallas import tpu as pltpu
from jax.experimental.pallas import tpu_sc as plsc

N = 10000
E = 320000
D_IN = 128
H = 32
C = 40

NC = 2
NS = 16
L = 16
NW = NC * NS

NP = 10240
NPT = NP // NS

CHUNK = 256
EPT = 10240
REAL_EPT = E // NW
CPT = EPT // CHUNK
NSLOT = 4

BLK = 2000
GRID = N // BLK

_MESH = dict(core_axis_name="c", subcore_axis_name="s")


DW = 8


@functools.partial(
    pl.kernel,
    out_type=jax.ShapeDtypeStruct((NC, NP, DW), jnp.float32),
    mesh=plsc.VectorSubcoreMesh(**_MESH),
    compiler_params=pltpu.CompilerParams(use_tc_tiling_on_sc=False),
    scratch_types=[
        pltpu.VMEM((EPT,), jnp.int32),
        pltpu.VMEM((CHUNK, DW), jnp.float32),
        pltpu.VMEM((NPT, DW), jnp.float32),
        pltpu.VMEM_SHARED((NP, DW), jnp.float32),
        pltpu.SemaphoreType.DMA,
    ],
)
def _deg_kernel(ei_hbm, ones_hbm, zeros_hbm, out_hbm,
                didx, ones_v, zbuf, acc_sh, sem):
    c = lax.axis_index("c")
    s = lax.axis_index("s")
    wid = s * NC + c
    pltpu.sync_copy(ones_hbm, ones_v)
    pltpu.sync_copy(zeros_hbm, zbuf)
    pltpu.sync_copy(zbuf, acc_sh.at[pl.ds(s * NPT, NPT)])
    pltpu.sync_copy(ei_hbm.at[1, pl.ds(wid * REAL_EPT, REAL_EPT)],
                    didx.at[pl.ds(0, REAL_EPT)])
    padv = jnp.full((L,), N, jnp.int32)

    def fbody(i, carry):
        didx[pl.ds(REAL_EPT + i * L, L)] = padv
        return carry

    lax.fori_loop(0, (EPT - REAL_EPT) // L, fbody, 0)
    plsc.subcore_barrier()

    def chunk(j, carry):
        pltpu.async_copy(ones_v, acc_sh.at[didx.at[pl.ds(j * CHUNK, CHUNK)]],
                         sem, add=True)
        return carry

    lax.fori_loop(0, CPT, chunk, 0)

    def drain(j, carry):
        pltpu.make_async_copy(
            ones_v, acc_sh.at[didx.at[pl.ds(j * CHUNK, CHUNK)]], sem).wait()
        return carry

    lax.fori_loop(0, CPT, drain, 0)
    plsc.subcore_barrier()
    pltpu.sync_copy(acc_sh.at[pl.ds(s * NPT, NPT)], zbuf)
    pltpu.sync_copy(zbuf, out_hbm.at[c, pl.ds(s * NPT, NPT)])


@functools.partial(
    pl.kernel,
    out_type=jax.ShapeDtypeStruct((NC, NP, H), jnp.float32),
    mesh=plsc.VectorSubcoreMesh(**_MESH),
    compiler_params=pltpu.CompilerParams(use_tc_tiling_on_sc=False),
    scratch_types=(
        [pltpu.VMEM((EPT,), jnp.int32),
         pltpu.VMEM((EPT,), jnp.int32)]
        + [pltpu.VMEM((CHUNK, H), jnp.float32) for _ in range(NSLOT)]
        + [pltpu.VMEM((NPT, H), jnp.float32)]
        + [pltpu.VMEM_SHARED((NP, H), jnp.float32),
           pltpu.VMEM_SHARED((NP, H), jnp.float32)]
        + [pltpu.SemaphoreType.DMA for _ in range(2 * NSLOT)]
    ),
)
def _scatter_kernel(hs_hbm, ei_hbm, out_hbm, sidx, didx, *rest):
    rows = rest[:NSLOT]
    zbuf = rest[NSLOT]
    acc_sh = rest[NSLOT + 1]
    hs_sh = rest[NSLOT + 2]
    gsem = rest[NSLOT + 3:2 * NSLOT + 3]
    ssem = rest[2 * NSLOT + 3:]
    c = lax.axis_index("c")
    s = lax.axis_index("s")
    wid = s * NC + c
    zero = jnp.zeros((L,), jnp.float32)

    def zbody(i, carry):
        zbuf[i, pl.ds(0, L)] = zero
        zbuf[i, pl.ds(L, L)] = zero
        return carry

    lax.fori_loop(0, NPT, zbody, 0)
    pltpu.sync_copy(zbuf, acc_sh.at[pl.ds(s * NPT, NPT)])
    pltpu.sync_copy(hs_hbm.at[pl.ds(s * NPT, NPT)],
                    hs_sh.at[pl.ds(s * NPT, NPT)])
    pltpu.sync_copy(ei_hbm.at[0, pl.ds(wid * REAL_EPT, REAL_EPT)],
                    sidx.at[pl.ds(0, REAL_EPT)])
    pltpu.sync_copy(ei_hbm.at[1, pl.ds(wid * REAL_EPT, REAL_EPT)],
                    didx.at[pl.ds(0, REAL_EPT)])
    padv = jnp.full((L,), N, jnp.int32)

    def fbody(i, carry):
        sidx[pl.ds(REAL_EPT + i * L, L)] = padv
        didx[pl.ds(REAL_EPT + i * L, L)] = padv
        return carry

    lax.fori_loop(0, (EPT - REAL_EPT) // L, fbody, 0)
    plsc.subcore_barrier()

    def soff(cc):
        return sidx.at[pl.ds(cc * CHUNK, CHUNK)]

    def doff(cc):
        return didx.at[pl.ds(cc * CHUNK, CHUNK)]

    for b in range(NSLOT):
        pltpu.async_copy(hs_sh.at[soff(b)], rows[b], gsem[b])

    def chunk(j, carry):
        base = j * NSLOT
        for b in range(NSLOT):
            pltpu.make_async_copy(hs_sh.at[soff(base + b)], rows[b],
                                  gsem[b]).wait()
            pltpu.async_copy(rows[b], acc_sh.at[doff(base + b)], ssem[b],
                             add=True)

        @pl.when(j < CPT // NSLOT - 1)
        def _():
            for b in range(NSLOT):
                pltpu.make_async_copy(rows[b], acc_sh.at[doff(base + b)],
                                      ssem[b]).wait()
                pltpu.async_copy(hs_sh.at[soff(base + b + NSLOT)], rows[b],
                                 gsem[b])

        return carry

    lax.fori_loop(0, CPT // NSLOT, chunk, 0)
    for b in range(NSLOT):
        pltpu.make_async_copy(rows[b], acc_sh.at[doff(CPT - NSLOT + b)],
                              ssem[b]).wait()
    plsc.subcore_barrier()
    pltpu.sync_copy(acc_sh.at[pl.ds(s * NPT, NPT)], zbuf)
    pltpu.sync_copy(zbuf, out_hbm.at[c, pl.ds(s * NPT, NPT)])


BLK_A = 2048
GRID_A = NP // BLK_A


def _stage_a_body(x_ref, w_ref, deg_ref, hs_ref, dinv_ref):
    h = jnp.dot(x_ref[...], w_ref[...], preferred_element_type=jnp.float32)
    deg_col = deg_ref[0, :, 0:1] + deg_ref[1, :, 0:1]
    dinv = lax.rsqrt(deg_col + 1.0)
    hs_ref[...] = h * dinv
    dinv_ref[...] = jnp.broadcast_to(dinv, (BLK_A, H))


def _stage_a(xp, W1, degp):
    return pl.pallas_call(
        _stage_a_body,
        grid=(GRID_A,),
        in_specs=[
            pl.BlockSpec((BLK_A, D_IN), lambda i: (i, 0)),
            pl.BlockSpec((D_IN, H), lambda i: (0, 0)),
            pl.BlockSpec((NC, BLK_A, DW), lambda i: (0, i, 0)),
        ],
        out_specs=[
            pl.BlockSpec((BLK_A, H), lambda i: (i, 0)),
            pl.BlockSpec((BLK_A, H), lambda i: (i, 0)),
        ],
        out_shape=[
            jax.ShapeDtypeStruct((NP, H), jnp.float32),
            jax.ShapeDtypeStruct((NP, H), jnp.float32),
        ],
    )(xp, W1, degp)


def _stage_bc_body(p_ref, hs_ref, dinv_ref, b_ref, g_ref, bln_ref, w_ref,
                   out_ref):
    dinv = dinv_ref[...]
    e = dinv * (p_ref[0] + p_ref[1] + hs_ref[...]) + b_ref[...]
    r = jnp.maximum(e, 0.0)
    m = jnp.mean(r, axis=1, keepdims=True)
    v = jnp.mean((r - m) ** 2, axis=1, keepdims=True)
    ln = (r - m) / jnp.sqrt(v + 1e-5) * g_ref[...] + bln_ref[...]
    out_ref[...] = jnp.dot(ln, w_ref[...],
                           preferred_element_type=jnp.float32) * dinv


def _stage_bc(parts, hs, dinvb, bias, g, bln, Wn):
    return pl.pallas_call(
        _stage_bc_body,
        grid=(GRID,),
        in_specs=[
            pl.BlockSpec((NC, BLK, H), lambda i: (0, i, 0)),
            pl.BlockSpec((BLK, H), lambda i: (i, 0)),
            pl.BlockSpec((BLK, H), lambda i: (i, 0)),
            pl.BlockSpec((1, H), lambda i: (0, 0)),
            pl.BlockSpec((1, H), lambda i: (0, 0)),
            pl.BlockSpec((1, H), lambda i: (0, 0)),
            pl.BlockSpec((H, H), lambda i: (0, 0)),
        ],
        out_specs=pl.BlockSpec((BLK, H), lambda i: (i, 0)),
        out_shape=jax.ShapeDtypeStruct((NP, H), jnp.float32),
    )(parts, hs, dinvb, bias.reshape(1, H), g.reshape(1, H),
      bln.reshape(1, H), Wn)


def _stage_d_body(p_ref, hs_ref, dinv_ref, b_ref, w1_ref, b1_ref, w2_ref,
                  b2_ref, emb_ref, logp_ref):
    e = dinv_ref[...] * (p_ref[0] + p_ref[1] + hs_ref[...]) + b_ref[...]
    emb_ref[...] = e
    r = jnp.maximum(e, 0.0)
    h1 = jnp.dot(r, w1_ref[...], preferred_element_type=jnp.float32) + b1_ref[...]
    h2 = jnp.dot(h1, w2_ref[...], preferred_element_type=jnp.float32) + b2_ref[...]
    m = jnp.max(h2, axis=1, keepdims=True)
    lse = jnp.log(jnp.sum(jnp.exp(h2 - m), axis=1, keepdims=True)) + m
    logp_ref[...] = h2 - lse


def _stage_d(parts, hs, dinvb, bias, mp1_W, mp1_b, mp2_W, mp2_b):
    return pl.pallas_call(
        _stage_d_body,
        grid=(GRID,),
        in_specs=[
            pl.BlockSpec((NC, BLK, H), lambda i: (0, i, 0)),
            pl.BlockSpec((BLK, H), lambda i: (i, 0)),
            pl.BlockSpec((BLK, H), lambda i: (i, 0)),
            pl.BlockSpec((1, H), lambda i: (0, 0)),
            pl.BlockSpec((H, H), lambda i: (0, 0)),
            pl.BlockSpec((1, H), lambda i: (0, 0)),
            pl.BlockSpec((H, C), lambda i: (0, 0)),
            pl.BlockSpec((1, C), lambda i: (0, 0)),
        ],
        out_specs=[
            pl.BlockSpec((BLK, H), lambda i: (i, 0)),
            pl.BlockSpec((BLK, C), lambda i: (i, 0)),
        ],
        out_shape=[
            jax.ShapeDtypeStruct((N, H), jnp.float32),
            jax.ShapeDtypeStruct((N, C), jnp.float32),
        ],
    )(parts, hs, dinvb, bias.reshape(1, H), mp1_W, mp1_b.reshape(1, H),
      mp2_W, mp2_b.reshape(1, C))


def kernel(x, edge_index, W1, b1, W2, b2, W3, b3, ln1_g, ln1_b, ln2_g, ln2_b,
           mp1_W, mp1_b, mp2_W, mp2_b):
    degp = _deg_kernel(edge_index,
                       jnp.ones((CHUNK, DW), jnp.float32),
                       jnp.zeros((NPT, DW), jnp.float32))
    xp = jnp.pad(x, ((0, NP - N), (0, 0)))
    hs1, dinvb = _stage_a(xp, W1, degp)
    p1 = _scatter_kernel(hs1, edge_index)
    hs2 = _stage_bc(p1, hs1, dinvb, b1, ln1_g, ln1_b, W2)
    p2 = _scatter_kernel(hs2, edge_index)
    hs3 = _stage_bc(p2, hs2, dinvb, b2, ln2_g, ln2_b, W3)
    p3 = _scatter_kernel(hs3, edge_index)
    emb, logp = _stage_d(p3, hs3, dinvb, b3, mp1_W, mp1_b, mp2_W, mp2_b)
    return (emb, logp)


# speedup vs baseline: 1.0016x; 1.0016x over previous
import functools

import jax
import jax.numpy as jnp
from jax import lax
from jax.experimental import pallas as pl
from jax.experimental.pallas import tpu as pltpu
from jax.experimental.pallas import tpu_sc as plsc

N = 10000
E = 320000
D_IN = 128
H = 32
C = 40

NC = 2
NS = 16
L = 16
NW = NC * NS

NP = 10240
NPT = NP // NS

CHUNK = 256
EPT = 10240
REAL_EPT = E // NW
CPT = EPT // CHUNK
NSLOT = 4

BLK = 2000
GRID = N // BLK

_MESH = dict(core_axis_name="c", subcore_axis_name="s")


DW = 8


@functools.partial(
    pl.kernel,
    out_type=jax.ShapeDtypeStruct((NC, NP, DW), jnp.float32),
    mesh=plsc.VectorSubcoreMesh(**_MESH),
    compiler_params=pltpu.CompilerParams(use_tc_tiling_on_sc=False),
    scratch_types=[
        pltpu.VMEM((EPT,), jnp.int32),
        pltpu.VMEM((CHUNK, DW), jnp.float32),
        pltpu.VMEM((NPT, DW), jnp.float32),
        pltpu.VMEM_SHARED((NP, DW), jnp.float32),
        pltpu.SemaphoreType.DMA,
    ],
)
def _deg_kernel(ei_hbm, ones_hbm, zeros_hbm, out_hbm,
                didx, ones_v, zbuf, acc_sh, sem):
    c = lax.axis_index("c")
    s = lax.axis_index("s")
    wid = s * NC + c
    pltpu.sync_copy(ones_hbm, ones_v)
    pltpu.sync_copy(zeros_hbm, zbuf)
    pltpu.sync_copy(zbuf, acc_sh.at[pl.ds(s * NPT, NPT)])
    pltpu.sync_copy(ei_hbm.at[1, pl.ds(wid * REAL_EPT, REAL_EPT)],
                    didx.at[pl.ds(0, REAL_EPT)])
    padv = jnp.full((L,), N, jnp.int32)

    def fbody(i, carry):
        didx[pl.ds(REAL_EPT + i * L, L)] = padv
        return carry

    lax.fori_loop(0, (EPT - REAL_EPT) // L, fbody, 0)
    plsc.subcore_barrier()

    def chunk(j, carry):
        pltpu.async_copy(ones_v, acc_sh.at[didx.at[pl.ds(j * CHUNK, CHUNK)]],
                         sem, add=True)
        return carry

    lax.fori_loop(0, CPT, chunk, 0)

    def drain(j, carry):
        pltpu.make_async_copy(
            ones_v, acc_sh.at[didx.at[pl.ds(j * CHUNK, CHUNK)]], sem).wait()
        return carry

    lax.fori_loop(0, CPT, drain, 0)
    plsc.subcore_barrier()
    pltpu.sync_copy(acc_sh.at[pl.ds(s * NPT, NPT)], zbuf)
    pltpu.sync_copy(zbuf, out_hbm.at[c, pl.ds(s * NPT, NPT)])


@functools.partial(
    pl.kernel,
    out_type=jax.ShapeDtypeStruct((NC, NP, H), jnp.float32),
    mesh=plsc.VectorSubcoreMesh(**_MESH),
    compiler_params=pltpu.CompilerParams(use_tc_tiling_on_sc=False),
    scratch_types=(
        [pltpu.VMEM((EPT,), jnp.int32),
         pltpu.VMEM((EPT,), jnp.int32)]
        + [pltpu.VMEM((CHUNK, H), jnp.float32) for _ in range(NSLOT)]
        + [pltpu.VMEM((NPT, H), jnp.float32)]
        + [pltpu.VMEM_SHARED((NP, H), jnp.float32),
           pltpu.VMEM_SHARED((NP, H), jnp.float32)]
        + [pltpu.SemaphoreType.DMA for _ in range(2 * NSLOT)]
    ),
)
def _scatter_kernel(hs_hbm, ei_hbm, out_hbm, sidx, didx, *rest):
    rows = rest[:NSLOT]
    zbuf = rest[NSLOT]
    acc_sh = rest[NSLOT + 1]
    hs_sh = rest[NSLOT + 2]
    gsem = rest[NSLOT + 3:2 * NSLOT + 3]
    ssem = rest[2 * NSLOT + 3:]
    c = lax.axis_index("c")
    s = lax.axis_index("s")
    wid = s * NC + c
    zero = jnp.zeros((L,), jnp.float32)

    def zbody(i, carry):
        zbuf[i, pl.ds(0, L)] = zero
        zbuf[i, pl.ds(L, L)] = zero
        return carry

    lax.fori_loop(0, NPT, zbody, 0)
    pltpu.sync_copy(zbuf, acc_sh.at[pl.ds(s * NPT, NPT)])
    pltpu.sync_copy(hs_hbm.at[pl.ds(s * NPT, NPT)],
                    hs_sh.at[pl.ds(s * NPT, NPT)])
    pltpu.sync_copy(ei_hbm.at[0, pl.ds(wid * REAL_EPT, REAL_EPT)],
                    sidx.at[pl.ds(0, REAL_EPT)])
    pltpu.sync_copy(ei_hbm.at[1, pl.ds(wid * REAL_EPT, REAL_EPT)],
                    didx.at[pl.ds(0, REAL_EPT)])
    padv = jnp.full((L,), N, jnp.int32)

    def fbody(i, carry):
        sidx[pl.ds(REAL_EPT + i * L, L)] = padv
        didx[pl.ds(REAL_EPT + i * L, L)] = padv
        return carry

    lax.fori_loop(0, (EPT - REAL_EPT) // L, fbody, 0)
    plsc.subcore_barrier()

    def soff(cc):
        return sidx.at[pl.ds(cc * CHUNK, CHUNK)]

    def doff(cc):
        return didx.at[pl.ds(cc * CHUNK, CHUNK)]

    for b in range(NSLOT):
        pltpu.async_copy(hs_sh.at[soff(b)], rows[b], gsem[b])

    def chunk(j, carry):
        base = j * NSLOT
        for b in range(NSLOT):
            pltpu.make_async_copy(hs_sh.at[soff(base + b)], rows[b],
                                  gsem[b]).wait()
            pltpu.async_copy(rows[b], acc_sh.at[doff(base + b)], ssem[b],
                             add=True)

        @pl.when(j < CPT // NSLOT - 1)
        def _():
            for b in range(NSLOT):
                pltpu.make_async_copy(rows[b], acc_sh.at[doff(base + b)],
                                      ssem[b]).wait()
                pltpu.async_copy(hs_sh.at[soff(base + b + NSLOT)], rows[b],
                                 gsem[b])

        return carry

    lax.fori_loop(0, CPT // NSLOT, chunk, 0)
    for b in range(NSLOT):
        pltpu.make_async_copy(rows[b], acc_sh.at[doff(CPT - NSLOT + b)],
                              ssem[b]).wait()
    plsc.subcore_barrier()
    pltpu.sync_copy(acc_sh.at[pl.ds(s * NPT, NPT)],
                    out_hbm.at[c, pl.ds(s * NPT, NPT)])


BLK_A = 2048
GRID_A = NP // BLK_A


def _stage_a_body(x_ref, w_ref, deg_ref, hs_ref, dinv_ref):
    h = jnp.dot(x_ref[...], w_ref[...], preferred_element_type=jnp.float32)
    deg_col = deg_ref[0, :, 0:1] + deg_ref[1, :, 0:1]
    dinv = lax.rsqrt(deg_col + 1.0)
    hs_ref[...] = h * dinv
    dinv_ref[...] = jnp.broadcast_to(dinv, (BLK_A, H))


def _stage_a(xp, W1, degp):
    return pl.pallas_call(
        _stage_a_body,
        grid=(GRID_A,),
        in_specs=[
            pl.BlockSpec((BLK_A, D_IN), lambda i: (i, 0)),
            pl.BlockSpec((D_IN, H), lambda i: (0, 0)),
            pl.BlockSpec((NC, BLK_A, DW), lambda i: (0, i, 0)),
        ],
        out_specs=[
            pl.BlockSpec((BLK_A, H), lambda i: (i, 0)),
            pl.BlockSpec((BLK_A, H), lambda i: (i, 0)),
        ],
        out_shape=[
            jax.ShapeDtypeStruct((NP, H), jnp.float32),
            jax.ShapeDtypeStruct((NP, H), jnp.float32),
        ],
    )(xp, W1, degp)


def _stage_bc_body(p_ref, hs_ref, dinv_ref, b_ref, g_ref, bln_ref, w_ref,
                   out_ref):
    dinv = dinv_ref[...]
    e = dinv * (p_ref[0] + p_ref[1] + hs_ref[...]) + b_ref[...]
    r = jnp.maximum(e, 0.0)
    m = jnp.mean(r, axis=1, keepdims=True)
    v = jnp.mean((r - m) ** 2, axis=1, keepdims=True)
    ln = (r - m) / jnp.sqrt(v + 1e-5) * g_ref[...] + bln_ref[...]
    out_ref[...] = jnp.dot(ln, w_ref[...],
                           preferred_element_type=jnp.float32) * dinv


def _stage_bc(parts, hs, dinvb, bias, g, bln, Wn):
    return pl.pallas_call(
        _stage_bc_body,
        grid=(GRID,),
        in_specs=[
            pl.BlockSpec((NC, BLK, H), lambda i: (0, i, 0)),
            pl.BlockSpec((BLK, H), lambda i: (i, 0)),
            pl.BlockSpec((BLK, H), lambda i: (i, 0)),
            pl.BlockSpec((1, H), lambda i: (0, 0)),
            pl.BlockSpec((1, H), lambda i: (0, 0)),
            pl.BlockSpec((1, H), lambda i: (0, 0)),
            pl.BlockSpec((H, H), lambda i: (0, 0)),
        ],
        out_specs=pl.BlockSpec((BLK, H), lambda i: (i, 0)),
        out_shape=jax.ShapeDtypeStruct((NP, H), jnp.float32),
    )(parts, hs, dinvb, bias.reshape(1, H), g.reshape(1, H),
      bln.reshape(1, H), Wn)


def _stage_d_body(p_ref, hs_ref, dinv_ref, b_ref, w1_ref, b1_ref, w2_ref,
                  b2_ref, emb_ref, logp_ref):
    e = dinv_ref[...] * (p_ref[0] + p_ref[1] + hs_ref[...]) + b_ref[...]
    emb_ref[...] = e
    r = jnp.maximum(e, 0.0)
    h1 = jnp.dot(r, w1_ref[...], preferred_element_type=jnp.float32) + b1_ref[...]
    h2 = jnp.dot(h1, w2_ref[...], preferred_element_type=jnp.float32) + b2_ref[...]
    m = jnp.max(h2, axis=1, keepdims=True)
    lse = jnp.log(jnp.sum(jnp.exp(h2 - m), axis=1, keepdims=True)) + m
    logp_ref[...] = h2 - lse


def _stage_d(parts, hs, dinvb, bias, mp1_W, mp1_b, mp2_W, mp2_b):
    return pl.pallas_call(
        _stage_d_body,
        grid=(GRID,),
        in_specs=[
            pl.BlockSpec((NC, BLK, H), lambda i: (0, i, 0)),
            pl.BlockSpec((BLK, H), lambda i: (i, 0)),
            pl.BlockSpec((BLK, H), lambda i: (i, 0)),
            pl.BlockSpec((1, H), lambda i: (0, 0)),
            pl.BlockSpec((H, H), lambda i: (0, 0)),
            pl.BlockSpec((1, H), lambda i: (0, 0)),
            pl.BlockSpec((H, C), lambda i: (0, 0)),
            pl.BlockSpec((1, C), lambda i: (0, 0)),
        ],
        out_specs=[
            pl.BlockSpec((BLK, H), lambda i: (i, 0)),
            pl.BlockSpec((BLK, C), lambda i: (i, 0)),
        ],
        out_shape=[
            jax.ShapeDtypeStruct((N, H), jnp.float32),
            jax.ShapeDtypeStruct((N, C), jnp.float32),
        ],
    )(parts, hs, dinvb, bias.reshape(1, H), mp1_W, mp1_b.reshape(1, H),
      mp2_W, mp2_b.reshape(1, C))


def kernel(x, edge_index, W1, b1, W2, b2, W3, b3, ln1_g, ln1_b, ln2_g, ln2_b,
           mp1_W, mp1_b, mp2_W, mp2_b):
    degp = _deg_kernel(edge_index,
                       jnp.ones((CHUNK, DW), jnp.float32),
                       jnp.zeros((NPT, DW), jnp.float32))
    xp = jnp.pad(x, ((0, NP - N), (0, 0)))
    hs1, dinvb = _stage_a(xp, W1, degp)
    p1 = _scatter_kernel(hs1, edge_index)
    hs2 = _stage_bc(p1, hs1, dinvb, b1, ln1_g, ln1_b, W2)
    p2 = _scatter_kernel(hs2, edge_index)
    hs3 = _stage_bc(p2, hs2, dinvb, b2, ln2_g, ln2_b, W3)
    p3 = _scatter_kernel(hs3, edge_index)
    emb, logp = _stage_d(p3, hs3, dinvb, b3, mp1_W, mp1_b, mp2_W, mp2_b)
    return (emb, logp)


# speedup vs baseline: 1.0305x; 1.0289x over previous
import functools

import jax
import jax.numpy as jnp
from jax import lax
from jax.experimental import pallas as pl
from jax.experimental.pallas import tpu as pltpu
from jax.experimental.pallas import tpu_sc as plsc

N = 10000
E = 320000
D_IN = 128
H = 32
C = 40

NC = 2
NS = 16
L = 16
NW = NC * NS

NP = 10240
NPT = NP // NS

CHUNK = 256
EPT = 10240
REAL_EPT = E // NW
CPT = EPT // CHUNK
NSLOT = 4

BLK = 2000
GRID = N // BLK

_MESH = dict(core_axis_name="c", subcore_axis_name="s")


DW = 8


@functools.partial(
    pl.kernel,
    out_type=jax.ShapeDtypeStruct((NC, NP, DW), jnp.float32),
    mesh=plsc.VectorSubcoreMesh(**_MESH),
    compiler_params=pltpu.CompilerParams(use_tc_tiling_on_sc=False),
    scratch_types=[
        pltpu.VMEM((EPT,), jnp.int32),
        pltpu.VMEM((CHUNK, DW), jnp.float32),
        pltpu.VMEM((NPT, DW), jnp.float32),
        pltpu.VMEM_SHARED((NP, DW), jnp.float32),
        pltpu.SemaphoreType.DMA,
    ],
)
def _deg_kernel(ei_hbm, ones_hbm, zeros_hbm, out_hbm,
                didx, ones_v, zbuf, acc_sh, sem):
    c = lax.axis_index("c")
    s = lax.axis_index("s")
    wid = s * NC + c
    pltpu.sync_copy(ones_hbm, ones_v)
    pltpu.sync_copy(zeros_hbm, zbuf)
    pltpu.sync_copy(zbuf, acc_sh.at[pl.ds(s * NPT, NPT)])
    pltpu.sync_copy(ei_hbm.at[1, pl.ds(wid * REAL_EPT, REAL_EPT)],
                    didx.at[pl.ds(0, REAL_EPT)])
    padv = jnp.full((L,), N, jnp.int32)

    def fbody(i, carry):
        didx[pl.ds(REAL_EPT + i * L, L)] = padv
        return carry

    lax.fori_loop(0, (EPT - REAL_EPT) // L, fbody, 0)
    plsc.subcore_barrier()

    def chunk(j, carry):
        pltpu.async_copy(ones_v, acc_sh.at[didx.at[pl.ds(j * CHUNK, CHUNK)]],
                         sem, add=True)
        return carry

    lax.fori_loop(0, CPT, chunk, 0)

    def drain(j, carry):
        pltpu.make_async_copy(
            ones_v, acc_sh.at[didx.at[pl.ds(j * CHUNK, CHUNK)]], sem).wait()
        return carry

    lax.fori_loop(0, CPT, drain, 0)
    plsc.subcore_barrier()
    pltpu.sync_copy(acc_sh.at[pl.ds(s * NPT, NPT)], zbuf)
    pltpu.sync_copy(zbuf, out_hbm.at[c, pl.ds(s * NPT, NPT)])


@functools.partial(
    pl.kernel,
    out_type=jax.ShapeDtypeStruct((NC, NP, H), jnp.float32),
    mesh=plsc.VectorSubcoreMesh(**_MESH),
    compiler_params=pltpu.CompilerParams(use_tc_tiling_on_sc=False),
    scratch_types=(
        [pltpu.VMEM((EPT,), jnp.int32),
         pltpu.VMEM((EPT,), jnp.int32)]
        + [pltpu.VMEM((CHUNK, H), jnp.float32) for _ in range(NSLOT)]
        + [pltpu.VMEM_SHARED((NP, H), jnp.float32),
           pltpu.VMEM_SHARED((NP, H), jnp.float32)]
        + [pltpu.SemaphoreType.DMA for _ in range(2 * NSLOT)]
    ),
)
def _scatter_kernel(hs_hbm, ei_hbm, zeros_hbm, out_hbm, sidx, didx, *rest):
    rows = rest[:NSLOT]
    acc_sh = rest[NSLOT]
    hs_sh = rest[NSLOT + 1]
    gsem = rest[NSLOT + 2:2 * NSLOT + 2]
    ssem = rest[2 * NSLOT + 2:]
    c = lax.axis_index("c")
    s = lax.axis_index("s")
    wid = s * NC + c

    pz = pltpu.async_copy(zeros_hbm, acc_sh.at[pl.ds(s * NPT, NPT)], gsem[0])
    ph = pltpu.async_copy(hs_hbm.at[pl.ds(s * NPT, NPT)],
                          hs_sh.at[pl.ds(s * NPT, NPT)], gsem[1])
    ps = pltpu.async_copy(ei_hbm.at[0, pl.ds(wid * REAL_EPT, REAL_EPT)],
                          sidx.at[pl.ds(0, REAL_EPT)], gsem[2])
    pd = pltpu.async_copy(ei_hbm.at[1, pl.ds(wid * REAL_EPT, REAL_EPT)],
                          didx.at[pl.ds(0, REAL_EPT)], gsem[3])
    padv = jnp.full((L,), N, jnp.int32)

    def fbody(i, carry):
        sidx[pl.ds(REAL_EPT + i * L, L)] = padv
        didx[pl.ds(REAL_EPT + i * L, L)] = padv
        return carry

    lax.fori_loop(0, (EPT - REAL_EPT) // L, fbody, 0)
    pz.wait()
    ph.wait()
    ps.wait()
    pd.wait()
    plsc.subcore_barrier()

    def soff(cc):
        return sidx.at[pl.ds(cc * CHUNK, CHUNK)]

    def doff(cc):
        return didx.at[pl.ds(cc * CHUNK, CHUNK)]

    for b in range(NSLOT):
        pltpu.async_copy(hs_sh.at[soff(b)], rows[b], gsem[b])

    def chunk(j, carry):
        base = j * NSLOT
        for b in range(NSLOT):
            pltpu.make_async_copy(hs_sh.at[soff(base + b)], rows[b],
                                  gsem[b]).wait()
            pltpu.async_copy(rows[b], acc_sh.at[doff(base + b)], ssem[b],
                             add=True)

        @pl.when(j < CPT // NSLOT - 1)
        def _():
            for b in range(NSLOT):
                pltpu.make_async_copy(rows[b], acc_sh.at[doff(base + b)],
                                      ssem[b]).wait()
                pltpu.async_copy(hs_sh.at[soff(base + b + NSLOT)], rows[b],
                                 gsem[b])

        return carry

    lax.fori_loop(0, CPT // NSLOT, chunk, 0)
    for b in range(NSLOT):
        pltpu.make_async_copy(rows[b], acc_sh.at[doff(CPT - NSLOT + b)],
                              ssem[b]).wait()
    plsc.subcore_barrier()
    pltpu.sync_copy(acc_sh.at[pl.ds(s * NPT, NPT)],
                    out_hbm.at[c, pl.ds(s * NPT, NPT)])


BLK_A = 2048
GRID_A = NP // BLK_A


def _stage_a_body(x_ref, w_ref, deg_ref, hs_ref, dinv_ref):
    h = jnp.dot(x_ref[...], w_ref[...], preferred_element_type=jnp.float32)
    deg_col = deg_ref[0, :, 0:1] + deg_ref[1, :, 0:1]
    dinv = lax.rsqrt(deg_col + 1.0)
    hs_ref[...] = h * dinv
    dinv_ref[...] = jnp.broadcast_to(dinv, (BLK_A, H))


def _stage_a(xp, W1, degp):
    return pl.pallas_call(
        _stage_a_body,
        grid=(GRID_A,),
        in_specs=[
            pl.BlockSpec((BLK_A, D_IN), lambda i: (i, 0)),
            pl.BlockSpec((D_IN, H), lambda i: (0, 0)),
            pl.BlockSpec((NC, BLK_A, DW), lambda i: (0, i, 0)),
        ],
        out_specs=[
            pl.BlockSpec((BLK_A, H), lambda i: (i, 0)),
            pl.BlockSpec((BLK_A, H), lambda i: (i, 0)),
        ],
        out_shape=[
            jax.ShapeDtypeStruct((NP, H), jnp.float32),
            jax.ShapeDtypeStruct((NP, H), jnp.float32),
        ],
    )(xp, W1, degp)


def _stage_bc_body(p_ref, hs_ref, dinv_ref, b_ref, g_ref, bln_ref, w_ref,
                   out_ref):
    dinv = dinv_ref[...]
    e = dinv * (p_ref[0] + p_ref[1] + hs_ref[...]) + b_ref[...]
    r = jnp.maximum(e, 0.0)
    m = jnp.mean(r, axis=1, keepdims=True)
    v = jnp.mean((r - m) ** 2, axis=1, keepdims=True)
    ln = (r - m) / jnp.sqrt(v + 1e-5) * g_ref[...] + bln_ref[...]
    out_ref[...] = jnp.dot(ln, w_ref[...],
                           preferred_element_type=jnp.float32) * dinv


def _stage_bc(parts, hs, dinvb, bias, g, bln, Wn):
    return pl.pallas_call(
        _stage_bc_body,
        grid=(GRID,),
        in_specs=[
            pl.BlockSpec((NC, BLK, H), lambda i: (0, i, 0)),
            pl.BlockSpec((BLK, H), lambda i: (i, 0)),
            pl.BlockSpec((BLK, H), lambda i: (i, 0)),
            pl.BlockSpec((1, H), lambda i: (0, 0)),
            pl.BlockSpec((1, H), lambda i: (0, 0)),
            pl.BlockSpec((1, H), lambda i: (0, 0)),
            pl.BlockSpec((H, H), lambda i: (0, 0)),
        ],
        out_specs=pl.BlockSpec((BLK, H), lambda i: (i, 0)),
        out_shape=jax.ShapeDtypeStruct((NP, H), jnp.float32),
    )(parts, hs, dinvb, bias.reshape(1, H), g.reshape(1, H),
      bln.reshape(1, H), Wn)


def _stage_d_body(p_ref, hs_ref, dinv_ref, b_ref, w1_ref, b1_ref, w2_ref,
                  b2_ref, emb_ref, logp_ref):
    e = dinv_ref[...] * (p_ref[0] + p_ref[1] + hs_ref[...]) + b_ref[...]
    emb_ref[...] = e
    r = jnp.maximum(e, 0.0)
    h1 = jnp.dot(r, w1_ref[...], preferred_element_type=jnp.float32) + b1_ref[...]
    h2 = jnp.dot(h1, w2_ref[...], preferred_element_type=jnp.float32) + b2_ref[...]
    m = jnp.max(h2, axis=1, keepdims=True)
    lse = jnp.log(jnp.sum(jnp.exp(h2 - m), axis=1, keepdims=True)) + m
    logp_ref[...] = h2 - lse


def _stage_d(parts, hs, dinvb, bias, mp1_W, mp1_b, mp2_W, mp2_b):
    return pl.pallas_call(
        _stage_d_body,
        grid=(GRID,),
        in_specs=[
            pl.BlockSpec((NC, BLK, H), lambda i: (0, i, 0)),
            pl.BlockSpec((BLK, H), lambda i: (i, 0)),
            pl.BlockSpec((BLK, H), lambda i: (i, 0)),
            pl.BlockSpec((1, H), lambda i: (0, 0)),
            pl.BlockSpec((H, H), lambda i: (0, 0)),
            pl.BlockSpec((1, H), lambda i: (0, 0)),
            pl.BlockSpec((H, C), lambda i: (0, 0)),
            pl.BlockSpec((1, C), lambda i: (0, 0)),
        ],
        out_specs=[
            pl.BlockSpec((BLK, H), lambda i: (i, 0)),
            pl.BlockSpec((BLK, C), lambda i: (i, 0)),
        ],
        out_shape=[
            jax.ShapeDtypeStruct((N, H), jnp.float32),
            jax.ShapeDtypeStruct((N, C), jnp.float32),
        ],
    )(parts, hs, dinvb, bias.reshape(1, H), mp1_W, mp1_b.reshape(1, H),
      mp2_W, mp2_b.reshape(1, C))


def kernel(x, edge_index, W1, b1, W2, b2, W3, b3, ln1_g, ln1_b, ln2_g, ln2_b,
           mp1_W, mp1_b, mp2_W, mp2_b):
    degp = _deg_kernel(edge_index,
                       jnp.ones((CHUNK, DW), jnp.float32),
                       jnp.zeros((NPT, DW), jnp.float32))
    xp = jnp.pad(x, ((0, NP - N), (0, 0)))
    hs1, dinvb = _stage_a(xp, W1, degp)
    zrows = jnp.zeros((NPT, H), jnp.float32)
    p1 = _scatter_kernel(hs1, edge_index, zrows)
    hs2 = _stage_bc(p1, hs1, dinvb, b1, ln1_g, ln1_b, W2)
    p2 = _scatter_kernel(hs2, edge_index, zrows)
    hs3 = _stage_bc(p2, hs2, dinvb, b2, ln2_g, ln2_b, W3)
    p3 = _scatter_kernel(hs3, edge_index, zrows)
    emb, logp = _stage_d(p3, hs3, dinvb, b3, mp1_W, mp1_b, mp2_W, mp2_b)
    return (emb, logp)


# speedup vs baseline: 1.0441x; 1.0132x over previous
import functools

import jax
import jax.numpy as jnp
from jax import lax
from jax.experimental import pallas as pl
from jax.experimental.pallas import tpu as pltpu
from jax.experimental.pallas import tpu_sc as plsc

N = 10000
E = 320000
D_IN = 128
H = 32
C = 40

NC = 2
NS = 16
L = 16
NW = NC * NS

NP = 10240
NPT = NP // NS

CHUNK = 256
EPT = 10240
REAL_EPT = E // NW
CPT = EPT // CHUNK
NSLOT = 4

BLK = 5000
GRID = N // BLK

_MESH = dict(core_axis_name="c", subcore_axis_name="s")


DW = 8


@functools.partial(
    pl.kernel,
    out_type=jax.ShapeDtypeStruct((NC, NP, DW), jnp.float32),
    mesh=plsc.VectorSubcoreMesh(**_MESH),
    compiler_params=pltpu.CompilerParams(use_tc_tiling_on_sc=False),
    scratch_types=[
        pltpu.VMEM((EPT,), jnp.int32),
        pltpu.VMEM((CHUNK, DW), jnp.float32),
        pltpu.VMEM((NPT, DW), jnp.float32),
        pltpu.VMEM_SHARED((NP, DW), jnp.float32),
        pltpu.SemaphoreType.DMA,
    ],
)
def _deg_kernel(ei_hbm, ones_hbm, zeros_hbm, out_hbm,
                didx, ones_v, zbuf, acc_sh, sem):
    c = lax.axis_index("c")
    s = lax.axis_index("s")
    wid = s * NC + c
    pltpu.sync_copy(ones_hbm, ones_v)
    pltpu.sync_copy(zeros_hbm, zbuf)
    pltpu.sync_copy(zbuf, acc_sh.at[pl.ds(s * NPT, NPT)])
    pltpu.sync_copy(ei_hbm.at[1, pl.ds(wid * REAL_EPT, REAL_EPT)],
                    didx.at[pl.ds(0, REAL_EPT)])
    padv = jnp.full((L,), N, jnp.int32)

    def fbody(i, carry):
        didx[pl.ds(REAL_EPT + i * L, L)] = padv
        return carry

    lax.fori_loop(0, (EPT - REAL_EPT) // L, fbody, 0)
    plsc.subcore_barrier()

    def chunk(j, carry):
        pltpu.async_copy(ones_v, acc_sh.at[didx.at[pl.ds(j * CHUNK, CHUNK)]],
                         sem, add=True)
        return carry

    lax.fori_loop(0, CPT, chunk, 0)

    def drain(j, carry):
        pltpu.make_async_copy(
            ones_v, acc_sh.at[didx.at[pl.ds(j * CHUNK, CHUNK)]], sem).wait()
        return carry

    lax.fori_loop(0, CPT, drain, 0)
    plsc.subcore_barrier()
    pltpu.sync_copy(acc_sh.at[pl.ds(s * NPT, NPT)], zbuf)
    pltpu.sync_copy(zbuf, out_hbm.at[c, pl.ds(s * NPT, NPT)])


@functools.partial(
    pl.kernel,
    out_type=jax.ShapeDtypeStruct((NC, NP, H), jnp.float32),
    mesh=plsc.VectorSubcoreMesh(**_MESH),
    compiler_params=pltpu.CompilerParams(use_tc_tiling_on_sc=False),
    scratch_types=(
        [pltpu.VMEM((EPT,), jnp.int32),
         pltpu.VMEM((EPT,), jnp.int32)]
        + [pltpu.VMEM((CHUNK, H), jnp.float32) for _ in range(NSLOT)]
        + [pltpu.VMEM_SHARED((NP, H), jnp.float32),
           pltpu.VMEM_SHARED((NP, H), jnp.float32)]
        + [pltpu.SemaphoreType.DMA for _ in range(2 * NSLOT)]
    ),
)
def _scatter_kernel(hs_hbm, ei_hbm, zeros_hbm, out_hbm, sidx, didx, *rest):
    rows = rest[:NSLOT]
    acc_sh = rest[NSLOT]
    hs_sh = rest[NSLOT + 1]
    gsem = rest[NSLOT + 2:2 * NSLOT + 2]
    ssem = rest[2 * NSLOT + 2:]
    c = lax.axis_index("c")
    s = lax.axis_index("s")
    wid = s * NC + c

    pz = pltpu.async_copy(zeros_hbm, acc_sh.at[pl.ds(s * NPT, NPT)], gsem[0])
    ph = pltpu.async_copy(hs_hbm.at[pl.ds(s * NPT, NPT)],
                          hs_sh.at[pl.ds(s * NPT, NPT)], gsem[1])
    ps = pltpu.async_copy(ei_hbm.at[0, pl.ds(wid * REAL_EPT, REAL_EPT)],
                          sidx.at[pl.ds(0, REAL_EPT)], gsem[2])
    pd = pltpu.async_copy(ei_hbm.at[1, pl.ds(wid * REAL_EPT, REAL_EPT)],
                          didx.at[pl.ds(0, REAL_EPT)], gsem[3])
    padv = jnp.full((L,), N, jnp.int32)

    def fbody(i, carry):
        sidx[pl.ds(REAL_EPT + i * L, L)] = padv
        didx[pl.ds(REAL_EPT + i * L, L)] = padv
        return carry

    lax.fori_loop(0, (EPT - REAL_EPT) // L, fbody, 0)
    pz.wait()
    ph.wait()
    ps.wait()
    pd.wait()
    plsc.subcore_barrier()

    def soff(cc):
        return sidx.at[pl.ds(cc * CHUNK, CHUNK)]

    def doff(cc):
        return didx.at[pl.ds(cc * CHUNK, CHUNK)]

    for b in range(NSLOT):
        pltpu.async_copy(hs_sh.at[soff(b)], rows[b], gsem[b])

    def chunk(j, carry):
        base = j * NSLOT
        for b in range(NSLOT):
            pltpu.make_async_copy(hs_sh.at[soff(base + b)], rows[b],
                                  gsem[b]).wait()
            pltpu.async_copy(rows[b], acc_sh.at[doff(base + b)], ssem[b],
                             add=True)

        @pl.when(j < CPT // NSLOT - 1)
        def _():
            for b in range(NSLOT):
                pltpu.make_async_copy(rows[b], acc_sh.at[doff(base + b)],
                                      ssem[b]).wait()
                pltpu.async_copy(hs_sh.at[soff(base + b + NSLOT)], rows[b],
                                 gsem[b])

        return carry

    lax.fori_loop(0, CPT // NSLOT, chunk, 0)
    for b in range(NSLOT):
        pltpu.make_async_copy(rows[b], acc_sh.at[doff(CPT - NSLOT + b)],
                              ssem[b]).wait()
    plsc.subcore_barrier()
    pltpu.sync_copy(acc_sh.at[pl.ds(s * NPT, NPT)],
                    out_hbm.at[c, pl.ds(s * NPT, NPT)])


BLK_A = 2560
GRID_A = NP // BLK_A


def _stage_a_body(x_ref, w_ref, deg_ref, hs_ref, dinv_ref):
    h = jnp.dot(x_ref[...], w_ref[...], preferred_element_type=jnp.float32)
    deg_col = deg_ref[0, :, 0:1] + deg_ref[1, :, 0:1]
    dinv = lax.rsqrt(deg_col + 1.0)
    hs_ref[...] = h * dinv
    dinv_ref[...] = jnp.broadcast_to(dinv, (BLK_A, H))


def _stage_a(xp, W1, degp):
    return pl.pallas_call(
        _stage_a_body,
        grid=(GRID_A,),
        in_specs=[
            pl.BlockSpec((BLK_A, D_IN), lambda i: (i, 0)),
            pl.BlockSpec((D_IN, H), lambda i: (0, 0)),
            pl.BlockSpec((NC, BLK_A, DW), lambda i: (0, i, 0)),
        ],
        out_specs=[
            pl.BlockSpec((BLK_A, H), lambda i: (i, 0)),
            pl.BlockSpec((BLK_A, H), lambda i: (i, 0)),
        ],
        out_shape=[
            jax.ShapeDtypeStruct((NP, H), jnp.float32),
            jax.ShapeDtypeStruct((NP, H), jnp.float32),
        ],
    )(xp, W1, degp)


def _stage_bc_body(p_ref, hs_ref, dinv_ref, b_ref, g_ref, bln_ref, w_ref,
                   out_ref):
    dinv = dinv_ref[...]
    e = dinv * (p_ref[0] + p_ref[1] + hs_ref[...]) + b_ref[...]
    r = jnp.maximum(e, 0.0)
    m = jnp.mean(r, axis=1, keepdims=True)
    v = jnp.mean((r - m) ** 2, axis=1, keepdims=True)
    ln = (r - m) / jnp.sqrt(v + 1e-5) * g_ref[...] + bln_ref[...]
    out_ref[...] = jnp.dot(ln, w_ref[...],
                           preferred_element_type=jnp.float32) * dinv


def _stage_bc(parts, hs, dinvb, bias, g, bln, Wn):
    return pl.pallas_call(
        _stage_bc_body,
        grid=(GRID,),
        in_specs=[
            pl.BlockSpec((NC, BLK, H), lambda i: (0, i, 0)),
            pl.BlockSpec((BLK, H), lambda i: (i, 0)),
            pl.BlockSpec((BLK, H), lambda i: (i, 0)),
            pl.BlockSpec((1, H), lambda i: (0, 0)),
            pl.BlockSpec((1, H), lambda i: (0, 0)),
            pl.BlockSpec((1, H), lambda i: (0, 0)),
            pl.BlockSpec((H, H), lambda i: (0, 0)),
        ],
        out_specs=pl.BlockSpec((BLK, H), lambda i: (i, 0)),
        out_shape=jax.ShapeDtypeStruct((NP, H), jnp.float32),
    )(parts, hs, dinvb, bias.reshape(1, H), g.reshape(1, H),
      bln.reshape(1, H), Wn)


def _stage_d_body(p_ref, hs_ref, dinv_ref, b_ref, w1_ref, b1_ref, w2_ref,
                  b2_ref, emb_ref, logp_ref):
    e = dinv_ref[...] * (p_ref[0] + p_ref[1] + hs_ref[...]) + b_ref[...]
    emb_ref[...] = e
    r = jnp.maximum(e, 0.0)
    h1 = jnp.dot(r, w1_ref[...], preferred_element_type=jnp.float32) + b1_ref[...]
    h2 = jnp.dot(h1, w2_ref[...], preferred_element_type=jnp.float32) + b2_ref[...]
    m = jnp.max(h2, axis=1, keepdims=True)
    lse = jnp.log(jnp.sum(jnp.exp(h2 - m), axis=1, keepdims=True)) + m
    logp_ref[...] = h2 - lse


def _stage_d(parts, hs, dinvb, bias, mp1_W, mp1_b, mp2_W, mp2_b):
    return pl.pallas_call(
        _stage_d_body,
        grid=(GRID,),
        in_specs=[
            pl.BlockSpec((NC, BLK, H), lambda i: (0, i, 0)),
            pl.BlockSpec((BLK, H), lambda i: (i, 0)),
            pl.BlockSpec((BLK, H), lambda i: (i, 0)),
            pl.BlockSpec((1, H), lambda i: (0, 0)),
            pl.BlockSpec((H, H), lambda i: (0, 0)),
            pl.BlockSpec((1, H), lambda i: (0, 0)),
            pl.BlockSpec((H, C), lambda i: (0, 0)),
            pl.BlockSpec((1, C), lambda i: (0, 0)),
        ],
        out_specs=[
            pl.BlockSpec((BLK, H), lambda i: (i, 0)),
            pl.BlockSpec((BLK, C), lambda i: (i, 0)),
        ],
        out_shape=[
            jax.ShapeDtypeStruct((N, H), jnp.float32),
            jax.ShapeDtypeStruct((N, C), jnp.float32),
        ],
    )(parts, hs, dinvb, bias.reshape(1, H), mp1_W, mp1_b.reshape(1, H),
      mp2_W, mp2_b.reshape(1, C))


def kernel(x, edge_index, W1, b1, W2, b2, W3, b3, ln1_g, ln1_b, ln2_g, ln2_b,
           mp1_W, mp1_b, mp2_W, mp2_b):
    degp = _deg_kernel(edge_index,
                       jnp.ones((CHUNK, DW), jnp.float32),
                       jnp.zeros((NPT, DW), jnp.float32))
    xp = jnp.pad(x, ((0, NP - N), (0, 0)))
    hs1, dinvb = _stage_a(xp, W1, degp)
    zrows = jnp.zeros((NPT, H), jnp.float32)
    p1 = _scatter_kernel(hs1, edge_index, zrows)
    hs2 = _stage_bc(p1, hs1, dinvb, b1, ln1_g, ln1_b, W2)
    p2 = _scatter_kernel(hs2, edge_index, zrows)
    hs3 = _stage_bc(p2, hs2, dinvb, b2, ln2_g, ln2_b, W3)
    p3 = _scatter_kernel(hs3, edge_index, zrows)
    emb, logp = _stage_d(p3, hs3, dinvb, b3, mp1_W, mp1_b, mp2_W, mp2_b)
    return (emb, logp)
